# baseline (speedup 1.0000x reference)
import functools

import jax
import jax.numpy as jnp
from jax import lax
from jax.experimental import pallas as pl
from jax.experimental.pallas import tpu as pltpu
from jax.experimental.pallas import tpu_sc as plsc

N = 10000
D = 128
E = 320000
NTILE = 16
CHUNK = 120
K = 168
KB = 8
NBUF = 2
NSTAGE = K // KB
EPAD = NTILE * K * CHUNK
ROWS_PT = 632
NPAD = NTILE * ROWS_PT

_SPANS = []
_r = 0
while _r < ROWS_PT:
    _SPANS.append((_r, min(CHUNK, ROWS_PT - _r)))
    _r += CHUNK


def _sc_body(with_counts, *refs):
    if with_counts:
        (tab, s_all, d_all, zfeat, ones_hbm,
         agg, cnt,
         acc, sidx, didx, *bufsems) = refs
    else:
        (tab, s_all, d_all, zfeat,
         agg,
         acc, sidx, didx, *bufsems) = refs
        cnt = None
    bufs = bufsems[:NBUF]
    sems = bufsems[NBUF:]
    rows = bufs[0]

    core = lax.axis_index("c")
    sid = lax.axis_index("s")
    rbase = sid * ROWS_PT
    ebase = core * (NTILE * K) + sid * K
    obase = core * NPAD + rbase

    def zero_acc_slice():
        pltpu.sync_copy(zfeat, rows)
        for (o, l) in _SPANS:
            pltpu.sync_copy(rows.at[pl.ds(0, l)], acc.at[pl.ds(rbase + o, l)])

    def copy_out(dst_hbm):
        for (o, l) in _SPANS:
            pltpu.sync_copy(acc.at[pl.ds(rbase + o, l)], rows.at[pl.ds(0, l)])
            pltpu.sync_copy(rows.at[pl.ds(0, l)], dst_hbm.at[pl.ds(obase + o, l)])

    if with_counts:
        zero_acc_slice()
        pltpu.sync_copy(ones_hbm, rows)
        plsc.subcore_barrier()

        def cstage(s, carry):
            pltpu.sync_copy(d_all.at[pl.ds(ebase + s * KB, KB)], didx)

            def cbody(j, c2):
                pltpu.sync_copy(rows, acc.at[didx.at[j]], add=True)
                return c2

            return lax.fori_loop(0, KB, cbody, carry)

        lax.fori_loop(0, NSTAGE, cstage, 0)
        plsc.subcore_barrier()
        copy_out(cnt)

    zero_acc_slice()
    plsc.subcore_barrier()

    def pump(j, b):
        @pl.when(j + NBUF < KB)
        def _():
            pltpu.async_copy(tab.at[sidx.at[j + NBUF]], bufs[b], sems[b])

    def stage(s, carry):
        pltpu.sync_copy(s_all.at[pl.ds(ebase + s * KB, KB)], sidx)
        pltpu.sync_copy(d_all.at[pl.ds(ebase + s * KB, KB)], didx)
        for b in range(NBUF):
            pltpu.async_copy(tab.at[sidx.at[b]], bufs[b], sems[b])

        def body(t, c2):
            for b in range(NBUF):
                j = NBUF * t + b
                pltpu.make_async_copy(tab.at[sidx.at[j]],
                                      bufs[b], sems[b]).wait()
                pltpu.sync_copy(bufs[b], acc.at[didx.at[j]], add=True)
                pump(j, b)
            return c2

        return lax.fori_loop(0, KB // NBUF, body, carry)

    lax.fori_loop(0, NSTAGE, stage, 0)
    plsc.subcore_barrier()
    copy_out(agg)


def _make_sc_call(with_counts):
    n_out = 2 if with_counts else 1
    out_type = [jax.ShapeDtypeStruct((2 * NPAD, D), jnp.float32)] * n_out
    scratch = [
        pltpu.VMEM_SHARED((NPAD, D), jnp.float32),
        pltpu.VMEM((KB, CHUNK), jnp.int32),
        pltpu.VMEM((KB, CHUNK), jnp.int32),
    ]
    scratch += [pltpu.VMEM((CHUNK, D), jnp.float32)] * NBUF
    scratch += [pltpu.SemaphoreType.DMA] * NBUF
    mesh = plsc.VectorSubcoreMesh(core_axis_name="c", subcore_axis_name="s")
    return pl.kernel(
        functools.partial(_sc_body, with_counts),
        out_type=out_type,
        mesh=mesh,
        scratch_types=scratch,
    )


_sc_layer0 = _make_sc_call(True)
_sc_layer1 = _make_sc_call(False)


_TC_BLK = 1264
_TC_GRID = 2 * NPAD // _TC_BLK
_HALF = _TC_GRID // 2


def _tc_body(relu, agg_ref, cnt_ref, x_ref, wl_ref, wr_ref, b_ref, o_ref):
    c = jnp.maximum(cnt_ref[:, 0:1], 1.0)
    mean = agg_ref[...] / c
    acc = jnp.dot(mean, wl_ref[0], preferred_element_type=jnp.float32)
    acc = acc + jnp.dot(x_ref[...], wr_ref[0], preferred_element_type=jnp.float32)
    acc = acc + b_ref[0]
    if relu:
        acc = jnp.maximum(acc, 0.0)
    o_ref[...] = acc


def _tc_update(agg, cnt, xs, Wl2, Wr2, b2, relu, swap_out):
    if swap_out:
        out_spec = pl.BlockSpec((_TC_BLK, D),
                                lambda i: ((i + _HALF) % _TC_GRID, 0))
    else:
        out_spec = pl.BlockSpec((_TC_BLK, D), lambda i: (i, 0))
    return pl.pallas_call(
        functools.partial(_tc_body, relu),
        grid=(_TC_GRID,),
        in_specs=[
            pl.BlockSpec((_TC_BLK, D), lambda i: (i, 0)),
            pl.BlockSpec((_TC_BLK, D), lambda i: (i, 0)),
            pl.BlockSpec((_TC_BLK, D), lambda i: ((i + _HALF) % _TC_GRID, 0)),
            pl.BlockSpec((1, D, D), lambda i: (i // _HALF, 0, 0)),
            pl.BlockSpec((1, D, D), lambda i: (i // _HALF, 0, 0)),
            pl.BlockSpec((1, 1, D), lambda i: (i // _HALF, 0, 0)),
        ],
        out_specs=out_spec,
        out_shape=jax.ShapeDtypeStruct((2 * NPAD, D), jnp.float32),
    )(agg, cnt, xs, Wl2, Wr2, b2)


def _prep_edges(ei, src_off):
    src = ei[0].astype(jnp.int32) + src_off
    dst = ei[1].astype(jnp.int32)
    pad = EPAD - E
    src = jnp.concatenate([src, jnp.full((pad,), src_off, jnp.int32)])
    dst = jnp.concatenate([dst, jnp.full((pad,), N, jnp.int32)])
    return src.reshape(NTILE * K, CHUNK), dst.reshape(NTILE * K, CHUNK)


def kernel(x_user, x_item, ei_u2i, ei_i2u,
           W_l_l0_u2i, W_r_l0_u2i, b_l0_u2i,
           W_l_l0_i2u, W_r_l0_i2u, b_l0_i2u,
           W_l_l1_u2i, W_r_l1_u2i, b_l1_u2i,
           W_l_l1_i2u, W_r_l1_i2u, b_l1_i2u):
    s0, d0 = _prep_edges(ei_u2i, 0)
    s1, d1 = _prep_edges(ei_i2u, NPAD)
    s_all = jnp.concatenate([s0, s1])
    d_all = jnp.concatenate([d0, d1])
    zp = jnp.zeros((NPAD - N, D), jnp.float32)
    xs = jnp.concatenate([x_user, zp, x_item, zp])
    zfeat = jnp.zeros((CHUNK, D), jnp.float32)
    ones = jnp.ones((CHUNK, D), jnp.float32)
    Wl0 = jnp.stack([W_l_l0_u2i, W_l_l0_i2u])
    Wr0 = jnp.stack([W_r_l0_u2i, W_r_l0_i2u])
    b0 = jnp.stack([b_l0_u2i, b_l0_i2u])[:, None, :]
    Wl1 = jnp.stack([W_l_l1_u2i, W_l_l1_i2u])
    Wr1 = jnp.stack([W_r_l1_u2i, W_r_l1_i2u])
    b1 = jnp.stack([b_l1_u2i, b_l1_i2u])[:, None, :]

    agg0, cnt = _sc_layer0(xs, s_all, d_all, zfeat, ones)
    h = _tc_update(agg0, cnt, xs, Wl0, Wr0, b0, relu=True, swap_out=True)

    (agg1,) = _sc_layer1(h, s_all, d_all, zfeat)
    out = _tc_update(agg1, cnt, h, Wl1, Wr1, b1, relu=False, swap_out=False)
    return (out[NPAD:NPAD + N], out[:N])


# speedup vs baseline: 1.0533x; 1.0533x over previous
import functools

import jax
import jax.numpy as jnp
from jax import lax
from jax.experimental import pallas as pl
from jax.experimental.pallas import tpu as pltpu
from jax.experimental.pallas import tpu_sc as plsc

N = 10000
D = 128
E = 320000
NTILE = 16
CHUNK = 120
K = 168
KB = 8
NBUF = 2
NSTAGE = K // KB
EPAD = NTILE * K * CHUNK
ROWS_PT = 632
NPAD = NTILE * ROWS_PT

_SPANS = []
_r = 0
while _r < ROWS_PT:
    _SPANS.append((_r, min(CHUNK, ROWS_PT - _r)))
    _r += CHUNK


def _sc_body(with_counts, *refs):
    if with_counts:
        (tab, sd_all, zfeat, ones_hbm,
         agg, cnt,
         acc, sdidx, *bufsems) = refs
    else:
        (tab, sd_all, zfeat,
         agg,
         acc, sdidx, *bufsems) = refs
        cnt = None
    bufs = bufsems[:NBUF]
    sems = bufsems[NBUF:]
    rows = bufs[0]

    core = lax.axis_index("c")
    sid = lax.axis_index("s")
    rbase = sid * ROWS_PT
    gbase = (core * (NTILE * NSTAGE) + sid * NSTAGE) * 2 * KB
    obase = core * NPAD + rbase

    def sidx(j):
        return sdidx.at[j]

    def didx(j):
        return sdidx.at[KB + j]

    def zero_acc_slice():
        pltpu.sync_copy(zfeat, rows)
        for (o, l) in _SPANS:
            pltpu.sync_copy(rows.at[pl.ds(0, l)], acc.at[pl.ds(rbase + o, l)])

    def copy_out(dst_hbm):
        for (o, l) in _SPANS:
            pltpu.sync_copy(acc.at[pl.ds(rbase + o, l)], rows.at[pl.ds(0, l)])
            pltpu.sync_copy(rows.at[pl.ds(0, l)], dst_hbm.at[pl.ds(obase + o, l)])

    if with_counts:
        zero_acc_slice()
        pltpu.sync_copy(ones_hbm, rows)
        plsc.subcore_barrier()

        def cstage(s, carry):
            pltpu.sync_copy(sd_all.at[pl.ds(gbase + s * 2 * KB, 2 * KB)], sdidx)

            def cbody(j, c2):
                pltpu.sync_copy(rows, acc.at[didx(j)], add=True)
                return c2

            return lax.fori_loop(0, KB, cbody, carry)

        lax.fori_loop(0, NSTAGE, cstage, 0)
        plsc.subcore_barrier()
        copy_out(cnt)

    zero_acc_slice()
    plsc.subcore_barrier()

    def pump(j, b):
        @pl.when(j + NBUF < KB)
        def _():
            pltpu.async_copy(tab.at[sidx(j + NBUF)], bufs[b], sems[b])

    def stage(s, carry):
        pltpu.sync_copy(sd_all.at[pl.ds(gbase + s * 2 * KB, 2 * KB)], sdidx)
        for b in range(NBUF):
            pltpu.async_copy(tab.at[sidx(b)], bufs[b], sems[b])

        def body(t, c2):
            for b in range(NBUF):
                j = NBUF * t + b
                pltpu.make_async_copy(tab.at[sidx(j)],
                                      bufs[b], sems[b]).wait()
                pltpu.sync_copy(bufs[b], acc.at[didx(j)], add=True)
                pump(j, b)
            return c2

        return lax.fori_loop(0, KB // NBUF, body, carry)

    lax.fori_loop(0, NSTAGE, stage, 0)
    plsc.subcore_barrier()
    copy_out(agg)


def _make_sc_call(with_counts):
    n_out = 2 if with_counts else 1
    out_type = [jax.ShapeDtypeStruct((2 * NPAD, D), jnp.float32)] * n_out
    scratch = [
        pltpu.VMEM_SHARED((NPAD, D), jnp.float32),
        pltpu.VMEM((2 * KB, CHUNK), jnp.int32),
    ]
    scratch += [pltpu.VMEM((CHUNK, D), jnp.float32)] * NBUF
    scratch += [pltpu.SemaphoreType.DMA] * NBUF
    mesh = plsc.VectorSubcoreMesh(core_axis_name="c", subcore_axis_name="s")
    return pl.kernel(
        functools.partial(_sc_body, with_counts),
        out_type=out_type,
        mesh=mesh,
        scratch_types=scratch,
    )


_sc_layer0 = _make_sc_call(True)
_sc_layer1 = _make_sc_call(False)


_TC_BLK = 1264
_TC_GRID = 2 * NPAD // _TC_BLK
_HALF = _TC_GRID // 2


def _tc_body(relu, agg_ref, cnt_ref, x_ref, wl_ref, wr_ref, b_ref, o_ref):
    c = jnp.maximum(cnt_ref[:, 0:1], 1.0)
    mean = agg_ref[...] / c
    acc = jnp.dot(mean, wl_ref[0], preferred_element_type=jnp.float32)
    acc = acc + jnp.dot(x_ref[...], wr_ref[0], preferred_element_type=jnp.float32)
    acc = acc + b_ref[0]
    if relu:
        acc = jnp.maximum(acc, 0.0)
    o_ref[...] = acc


def _tc_update(agg, cnt, xs, Wl2, Wr2, b2, relu, swap_out):
    if swap_out:
        out_spec = pl.BlockSpec((_TC_BLK, D),
                                lambda i: ((i + _HALF) % _TC_GRID, 0))
    else:
        out_spec = pl.BlockSpec((_TC_BLK, D), lambda i: (i, 0))
    return pl.pallas_call(
        functools.partial(_tc_body, relu),
        grid=(_TC_GRID,),
        in_specs=[
            pl.BlockSpec((_TC_BLK, D), lambda i: (i, 0)),
            pl.BlockSpec((_TC_BLK, D), lambda i: (i, 0)),
            pl.BlockSpec((_TC_BLK, D), lambda i: ((i + _HALF) % _TC_GRID, 0)),
            pl.BlockSpec((1, D, D), lambda i: (i // _HALF, 0, 0)),
            pl.BlockSpec((1, D, D), lambda i: (i // _HALF, 0, 0)),
            pl.BlockSpec((1, 1, D), lambda i: (i // _HALF, 0, 0)),
        ],
        out_specs=out_spec,
        out_shape=jax.ShapeDtypeStruct((2 * NPAD, D), jnp.float32),
    )(agg, cnt, xs, Wl2, Wr2, b2)


def _prep_edges(ei, src_off):
    src = ei[0].astype(jnp.int32) + src_off
    dst = ei[1].astype(jnp.int32)
    pad = EPAD - E
    src = jnp.concatenate([src, jnp.full((pad,), src_off, jnp.int32)])
    dst = jnp.concatenate([dst, jnp.full((pad,), N, jnp.int32)])
    return src.reshape(NTILE * K, CHUNK), dst.reshape(NTILE * K, CHUNK)


def kernel(x_user, x_item, ei_u2i, ei_i2u,
           W_l_l0_u2i, W_r_l0_u2i, b_l0_u2i,
           W_l_l0_i2u, W_r_l0_i2u, b_l0_i2u,
           W_l_l1_u2i, W_r_l1_u2i, b_l1_u2i,
           W_l_l1_i2u, W_r_l1_i2u, b_l1_i2u):
    s0, d0 = _prep_edges(ei_u2i, 0)
    s1, d1 = _prep_edges(ei_i2u, NPAD)
    s_all = jnp.concatenate([s0, s1])
    d_all = jnp.concatenate([d0, d1])
    sd_all = jnp.concatenate(
        [s_all.reshape(-1, 1, KB, CHUNK), d_all.reshape(-1, 1, KB, CHUNK)],
        axis=1).reshape(-1, CHUNK)
    zp = jnp.zeros((NPAD - N, D), jnp.float32)
    xs = jnp.concatenate([x_user, zp, x_item, zp])
    zfeat = jnp.zeros((CHUNK, D), jnp.float32)
    ones = jnp.ones((CHUNK, D), jnp.float32)
    Wl0 = jnp.stack([W_l_l0_u2i, W_l_l0_i2u])
    Wr0 = jnp.stack([W_r_l0_u2i, W_r_l0_i2u])
    b0 = jnp.stack([b_l0_u2i, b_l0_i2u])[:, None, :]
    Wl1 = jnp.stack([W_l_l1_u2i, W_l_l1_i2u])
    Wr1 = jnp.stack([W_r_l1_u2i, W_r_l1_i2u])
    b1 = jnp.stack([b_l1_u2i, b_l1_i2u])[:, None, :]

    agg0, cnt = _sc_layer0(xs, sd_all, zfeat, ones)
    h = _tc_update(agg0, cnt, xs, Wl0, Wr0, b0, relu=True, swap_out=True)

    (agg1,) = _sc_layer1(h, sd_all, zfeat)
    out = _tc_update(agg1, cnt, h, Wl1, Wr1, b1, relu=False, swap_out=False)
    return (out[NPAD:NPAD + N], out[:N])
